# baseline (speedup 1.0000x reference)
import functools

import jax
import jax.numpy as jnp
from jax.experimental import pallas as pl
from jax.experimental.pallas import tpu as pltpu

N = 10000
D = 256
E = 160000
NP = 10240
ROWS = 1024


def _mm3_body(a0, a1, a2, w0, w1, w2, b, o_ref, *, relu):
    acc = (
        jnp.dot(a0[...], w0[...], preferred_element_type=jnp.float32)
        + jnp.dot(a1[...], w1[...], preferred_element_type=jnp.float32)
        + jnp.dot(a2[...], w2[...], preferred_element_type=jnp.float32)
        + b[...]
    )
    o_ref[...] = jnp.maximum(acc, 0.0) if relu else acc


def _mm3(a0, a1, a2, w0, w1, w2, b, relu):
    grid = (NP // ROWS,)
    blk = pl.BlockSpec((ROWS, D), lambda i: (i, 0))
    wspec = pl.BlockSpec((D, D), lambda i: (0, 0))
    bspec = pl.BlockSpec((1, D), lambda i: (0, 0))
    return pl.pallas_call(
        functools.partial(_mm3_body, relu=relu),
        grid=grid,
        in_specs=[blk, blk, blk, wspec, wspec, wspec, bspec],
        out_specs=blk,
        out_shape=jax.ShapeDtypeStruct((NP, D), jnp.float32),
    )(a0, a1, a2, w0, w1, w2, b)


def _agg(x, src, dst):
    ones = jnp.ones(src.shape[0], dtype=jnp.float32)
    deg_out = jnp.zeros((N,), jnp.float32).at[src].add(ones)
    deg_in = jnp.zeros((N,), jnp.float32).at[dst].add(ones)
    ns = jnp.where(deg_out > 0, deg_out ** -0.5, 0.0)
    nd = jnp.where(deg_in > 0, deg_in ** -0.5, 0.0)
    w_e = ns[src] * nd[dst]
    return jnp.zeros((N, D), jnp.float32).at[dst].add(x[src] * w_e[:, None])


def kernel(x, edge_index_r0, edge_index_r1, edge_index_r2, neg_edge_index, etype,
           W1_0, b1_0, W1_1, b1_1, W1_2, b1_2,
           W2_0, b2_0, W2_1, b2_1, W2_2, b2_2):
    edges = [edge_index_r0.astype(jnp.int32), edge_index_r1.astype(jnp.int32),
             edge_index_r2.astype(jnp.int32)]
    neg = neg_edge_index.astype(jnp.int32)

    def pad(a):
        return jnp.pad(a, ((0, NP - N), (0, 0)))

    a = [pad(_agg(x, e[0], e[1])) for e in edges]
    b1 = (b1_0 + b1_1 + b1_2)[None, :]
    h = _mm3(a[0], a[1], a[2], W1_0, W1_1, W1_2, b1, relu=True)[:N]

    a2 = [pad(_agg(h, e[0], e[1])) for e in edges]
    b2 = (b2_0 + b2_1 + b2_2)[None, :]
    h2 = _mm3(a2[0], a2[1], a2[2], W2_0, W2_1, W2_2, b2, relu=False)[:N]

    pe = jnp.stack(edges)[jnp.asarray(etype)]
    pos = jnp.sum(h2[pe[0]] * h2[pe[1]], axis=-1, keepdims=True)
    negs = jnp.sum(h2[neg[0]] * h2[neg[1]], axis=-1, keepdims=True)
    return pos, negs
